# baseline (speedup 1.0000x reference)
import functools

import jax
import jax.numpy as jnp
from jax.experimental import pallas as pl

_KNN = 16
_NBASIS = 16


_R = 1024


def _pad_lanes(x, n_pad):
    pad = n_pad - x.shape[-1]
    if pad == 0:
        return x
    cfg = [(0, 0)] * (x.ndim - 1) + [(0, pad)]
    return jnp.pad(x, cfg)


def _bf(x):
    return x.astype(jnp.bfloat16).astype(jnp.float32)


def _conv_body(xo, yo, tho, xn, yn, thn, vg, w1t, b1, w2t, b2, wft, bo, out,
               *, crop, relu):
    c_in = vg.shape[0]
    R = out.shape[-1]
    xov = xo[...]
    yov = yo[...]
    thov = tho[...]
    co = jnp.cos(thov)
    so = jnp.sin(thov)
    dx = xn[...] - xov
    dy = yn[...] - yov
    rx = co * dx + so * dy
    ry = -so * dx + co * dy
    dth = thn[...] - thov
    rx = _bf(rx)
    ry = _bf(ry)
    cd = _bf(jnp.cos(dth))
    sd = _bf(jnp.sin(dth))
    w1 = _bf(w1t[...])
    b1v = b1[...]
    s1 = (w1[:, 0:1, None] * rx[None] + w1[:, 1:2, None] * ry[None]
          + w1[:, 2:3, None] * cd[None] + w1[:, 3:4, None] * sd[None])
    h = jax.nn.relu(_bf(s1) + b1v[:, :, None])
    hf = h.reshape(32, _KNN * R).astype(jnp.bfloat16)
    bas = jnp.dot(w2t[...].astype(jnp.bfloat16), hf,
                  preferred_element_type=jnp.float32)
    bas = bas + b2[...]
    bas = bas.reshape(_NBASIS, _KNN, R)
    dist2 = rx * rx + ry * ry
    mask = (jnp.sqrt(dist2) <= crop).astype(jnp.float32)
    bas = bas * mask[None]
    vgv = vg[...]
    if c_in > 1:
        bas = _bf(bas)
        vgv = _bf(vgv)
    g = jnp.zeros((_NBASIS, c_in, R), jnp.float32)
    for k in range(_KNN):
        g = g + bas[:, k, :][:, None, :] * vgv[:, k, :][None, :, :]
    gf = _bf(g).reshape(_NBASIS * c_in, R).astype(jnp.bfloat16)
    o = jnp.dot(wft[...].astype(jnp.bfloat16), gf,
                preferred_element_type=jnp.float32)
    o = o * (1.0 / _KNN) + bo[...]
    if relu:
        o = jax.nn.relu(o)
    out[...] = o


def _conv_pallas(xo, yo, tho, xn, yn, thn, vg, p, crop, relu):
    n = xo.shape[-1]
    c_in = vg.shape[0]
    c_out = p['W'].shape[-1]
    R = _R
    n_pad = -(-n // R) * R
    grid = n_pad // R
    xo, yo, tho, xn, yn, thn, vg = (
        _pad_lanes(a, n_pad) for a in (xo, yo, tho, xn, yn, thn, vg))
    wft = p['W'].reshape(_NBASIS * c_in, c_out).T
    w1t = p['W1'].T
    b1 = p['b1'].reshape(32, 1)
    b2 = p['b2'].reshape(_NBASIS, 1)
    bo = p['b'].reshape(c_out, 1)
    col = lambda i: (0, i)
    col3 = lambda i: (0, 0, i)
    full = lambda i: (0, 0)
    return pl.pallas_call(
        functools.partial(_conv_body, crop=crop, relu=relu),
        grid=(grid,),
        in_specs=[
            pl.BlockSpec((1, R), col),
            pl.BlockSpec((1, R), col),
            pl.BlockSpec((1, R), col),
            pl.BlockSpec((_KNN, R), col),
            pl.BlockSpec((_KNN, R), col),
            pl.BlockSpec((_KNN, R), col),
            pl.BlockSpec((c_in, _KNN, R), col3),
            pl.BlockSpec((32, 4), full),
            pl.BlockSpec((32, 1), full),
            pl.BlockSpec((_NBASIS, 32), full),
            pl.BlockSpec((_NBASIS, 1), full),
            pl.BlockSpec((c_out, _NBASIS * c_in), full),
            pl.BlockSpec((c_out, 1), full),
        ],
        out_specs=pl.BlockSpec((c_out, R), col),
        out_shape=jax.ShapeDtypeStruct((c_out, n_pad), jnp.float32),
    )(xo, yo, tho, xn, yn, thn, vg, w1t, b1, p['W2'].T, b2, wft, bo)[:, :n]


def _bn_body(x, gamma, beta, out):
    xv = x[...]
    m = jnp.mean(xv, axis=1, keepdims=True)
    v = jnp.mean((xv - m) ** 2, axis=1, keepdims=True)
    out[...] = (xv - m) / jnp.sqrt(v + 1e-5) * gamma[...] + beta[...]


def _bn_pallas(x, p):
    c, n = x.shape
    return pl.pallas_call(
        _bn_body,
        in_specs=[
            pl.BlockSpec((c, n), lambda: (0, 0)),
            pl.BlockSpec((c, 1), lambda: (0, 0)),
            pl.BlockSpec((c, 1), lambda: (0, 0)),
        ],
        out_specs=pl.BlockSpec((c, n), lambda: (0, 0)),
        out_shape=jax.ShapeDtypeStruct((c, n), jnp.float32),
    )(x, p['gamma'].reshape(c, 1), p['beta'].reshape(c, 1))


def _knn(p_out, p_in, chunk=2048):
    outs = []
    for s in range(0, p_out.shape[0], chunk):
        po = p_out[s:s + chunk, :2]
        d2 = jnp.sum((po[:, None, :] - p_in[None, :, :2]) ** 2, axis=-1)
        outs.append(jax.lax.top_k(-d2, _KNN)[1])
    return jnp.concatenate(outs, axis=0)


def _do_conv(p, points_in, values_t, points_out, crop, uinv, relu):
    if uinv is None:
        idx = _knn(points_out, points_in)
        idxt = idx.T
        xn = points_in[:, 0][idxt]
        yn = points_in[:, 1][idxt]
        thn = points_in[:, 2][idxt]
        uinv = (idxt, xn, yn, thn,
                points_out[:, 0][None], points_out[:, 1][None],
                points_out[:, 2][None])
    idxt, xn, yn, thn, xo, yo, tho = uinv
    vg = values_t[:, idxt]
    out = _conv_pallas(xo, yo, tho, xn, yn, thn, vg, p, crop, relu)
    return uinv, out


def kernel(values, input_points, points0, points1, params):
    radii = [0.2, 0.2 * (2.0 ** 0.5)]
    points_list = [points0, points1]
    points = points_list[0]
    _, v = _do_conv(params['first'], input_points, values.T, points, radii[0],
                    None, True)
    for i in range(2):
        v_in = _bn_pallas(v, params['bn'][i][0])
        uinv, v = _do_conv(params['blocks'][i][0], points, v_in, points,
                           radii[i], None, True)
        for j in range(1, 3):
            v = _bn_pallas(v, params['bn'][i][j])
            _, v = _do_conv(params['blocks'][i][j], points, v, None,
                            radii[i], uinv, True)
        v = v + v_in
        if i < 1:
            nxt = points_list[i + 1]
            v = _bn_pallas(v, params['bn'][i][3])
            _, v = _do_conv(params['down'][i], points, v, nxt, radii[i + 1],
                            None, True)
            points = nxt
    v = _bn_pallas(v, params['last_bn'])
    _, v = _do_conv(params['last'], points, v, points, radii[-1], None, False)
    return jnp.mean(v, axis=1)


# speedup vs baseline: 1.6482x; 1.6482x over previous
import functools

import jax
import jax.numpy as jnp
from jax.experimental import pallas as pl

_KNN = 16
_NBASIS = 16


_R = 1024


def _pad_lanes(x, n_pad):
    pad = n_pad - x.shape[-1]
    if pad == 0:
        return x
    cfg = [(0, 0)] * (x.ndim - 1) + [(0, pad)]
    return jnp.pad(x, cfg)


def _bf(x):
    return x.astype(jnp.bfloat16).astype(jnp.float32)


def _conv_body(xo, yo, tho, xn, yn, thn, vg, w1t, b1, w2t, b2, wft, bo, out,
               *, crop, relu):
    c_in = vg.shape[0]
    R = out.shape[-1]
    xov = xo[...]
    yov = yo[...]
    thov = tho[...]
    co = jnp.cos(thov)
    so = jnp.sin(thov)
    dx = xn[...] - xov
    dy = yn[...] - yov
    rx = co * dx + so * dy
    ry = -so * dx + co * dy
    dth = thn[...] - thov
    rx = _bf(rx)
    ry = _bf(ry)
    cd = _bf(jnp.cos(dth))
    sd = _bf(jnp.sin(dth))
    w1 = _bf(w1t[...])
    b1v = b1[...]
    s1 = (w1[:, 0:1, None] * rx[None] + w1[:, 1:2, None] * ry[None]
          + w1[:, 2:3, None] * cd[None] + w1[:, 3:4, None] * sd[None])
    h = jax.nn.relu(_bf(s1) + b1v[:, :, None])
    hf = h.reshape(32, _KNN * R).astype(jnp.bfloat16)
    bas = jnp.dot(w2t[...].astype(jnp.bfloat16), hf,
                  preferred_element_type=jnp.float32)
    bas = bas + b2[...]
    bas = bas.reshape(_NBASIS, _KNN, R)
    dist2 = rx * rx + ry * ry
    mask = (jnp.sqrt(dist2) <= crop).astype(jnp.float32)
    bas = bas * mask[None]
    vgv = vg[...]
    if c_in > 1:
        bas = _bf(bas)
        vgv = _bf(vgv)
    g = jnp.zeros((_NBASIS, c_in, R), jnp.float32)
    for k in range(_KNN):
        g = g + bas[:, k, :][:, None, :] * vgv[:, k, :][None, :, :]
    gf = _bf(g).reshape(_NBASIS * c_in, R).astype(jnp.bfloat16)
    o = jnp.dot(wft[...].astype(jnp.bfloat16), gf,
                preferred_element_type=jnp.float32)
    o = o * (1.0 / _KNN) + bo[...]
    if relu:
        o = jax.nn.relu(o)
    out[...] = o


def _conv_pallas(xo, yo, tho, xn, yn, thn, vg, p, crop, relu):
    n = xo.shape[-1]
    c_in = vg.shape[0]
    c_out = p['W'].shape[-1]
    R = _R
    n_pad = -(-n // R) * R
    grid = n_pad // R
    xo, yo, tho, xn, yn, thn, vg = (
        _pad_lanes(a, n_pad) for a in (xo, yo, tho, xn, yn, thn, vg))
    wft = p['W'].reshape(_NBASIS * c_in, c_out).T
    w1t = p['W1'].T
    b1 = p['b1'].reshape(32, 1)
    b2 = p['b2'].reshape(_NBASIS, 1)
    bo = p['b'].reshape(c_out, 1)
    col = lambda i: (0, i)
    col3 = lambda i: (0, 0, i)
    full = lambda i: (0, 0)
    return pl.pallas_call(
        functools.partial(_conv_body, crop=crop, relu=relu),
        grid=(grid,),
        in_specs=[
            pl.BlockSpec((1, R), col),
            pl.BlockSpec((1, R), col),
            pl.BlockSpec((1, R), col),
            pl.BlockSpec((_KNN, R), col),
            pl.BlockSpec((_KNN, R), col),
            pl.BlockSpec((_KNN, R), col),
            pl.BlockSpec((c_in, _KNN, R), col3),
            pl.BlockSpec((32, 4), full),
            pl.BlockSpec((32, 1), full),
            pl.BlockSpec((_NBASIS, 32), full),
            pl.BlockSpec((_NBASIS, 1), full),
            pl.BlockSpec((c_out, _NBASIS * c_in), full),
            pl.BlockSpec((c_out, 1), full),
        ],
        out_specs=pl.BlockSpec((c_out, R), col),
        out_shape=jax.ShapeDtypeStruct((c_out, n_pad), jnp.float32),
    )(xo, yo, tho, xn, yn, thn, vg, w1t, b1, p['W2'].T, b2, wft, bo)[:, :n]


def _bn_body(x, gamma, beta, out):
    xv = x[...]
    m = jnp.mean(xv, axis=1, keepdims=True)
    v = jnp.mean((xv - m) ** 2, axis=1, keepdims=True)
    out[...] = (xv - m) / jnp.sqrt(v + 1e-5) * gamma[...] + beta[...]


def _bn_pallas(x, p):
    c, n = x.shape
    return pl.pallas_call(
        _bn_body,
        in_specs=[
            pl.BlockSpec((c, n), lambda: (0, 0)),
            pl.BlockSpec((c, 1), lambda: (0, 0)),
            pl.BlockSpec((c, 1), lambda: (0, 0)),
        ],
        out_specs=pl.BlockSpec((c, n), lambda: (0, 0)),
        out_shape=jax.ShapeDtypeStruct((c, n), jnp.float32),
    )(x, p['gamma'].reshape(c, 1), p['beta'].reshape(c, 1))


_RQ = 512
_CH = 512
_INF = 1e30
_BIGI = (1 << 30)


def _knn_body(clo_ref, chi_ref, qx, qy, cx, cy, out_i, *, nipad):
    b = pl.program_id(0)
    clo = clo_ref[b]
    chi = chi_ref[b]
    qxv = qx[...]
    qyv = qy[...]
    it = jax.lax.broadcasted_iota(jnp.int32, (_RQ, _CH), 1)
    it32 = jax.lax.broadcasted_iota(jnp.int32, (_RQ, 2 * _KNN), 1)

    def chunk_step(j, carry):
        best_d, best_i = carry
        cxv = cx[j]
        cyv = cy[j]
        dx = qxv - cxv
        dy = qyv - cyv
        d2 = dx * dx + dy * dy
        ms, gis = [], []
        for _ in range(_KNN):
            m = jnp.min(d2, axis=1, keepdims=True)
            pos = jnp.min(jnp.where(d2 == m, it, _BIGI), axis=1, keepdims=True)
            ms.append(m)
            gis.append(pos + j * _CH)
            d2 = jnp.where(it == pos, _INF, d2)
        cd = jnp.concatenate(ms, axis=1)
        ci = jnp.concatenate(gis, axis=1)
        md = jnp.concatenate([best_d, cd], axis=1)
        mi = jnp.concatenate([best_i, ci], axis=1)
        nd, ni = [], []
        for _ in range(_KNN):
            m = jnp.min(md, axis=1, keepdims=True)
            pos = jnp.min(jnp.where(md == m, it32, _BIGI), axis=1,
                          keepdims=True)
            sel = jnp.min(jnp.where(it32 == pos, mi, _BIGI), axis=1,
                          keepdims=True)
            nd.append(m)
            ni.append(sel)
            md = jnp.where(it32 == pos, _INF, md)
        return (jnp.concatenate(nd, axis=1), jnp.concatenate(ni, axis=1))

    init = (jnp.full((_RQ, _KNN), _INF, jnp.float32),
            jnp.full((_RQ, _KNN), nipad - 1, jnp.int32))
    best_d, best_i = jax.lax.fori_loop(clo, chi, chunk_step, init)
    out_i[...] = best_i


def _knn_pallas(sqx, sqy, scx, scy, margin):
    from jax.experimental.pallas import tpu as pltpu
    nq = sqx.shape[0]
    nc = scx.shape[0]
    nqp = -(-nq // _RQ) * _RQ
    ncp = -(-nc // _CH) * _CH
    nblocks = nqp // _RQ
    nchunks = ncp // _CH
    qpad = sqx[-1]
    sqxp = jnp.concatenate([sqx, jnp.full((nqp - nq,), qpad, jnp.float32)])
    sqyp = _pad_lanes(sqy, nqp)
    scxp = jnp.concatenate([scx, jnp.full((ncp - nc,), 1e4, jnp.float32)])
    scyp = jnp.concatenate([scy, jnp.full((ncp - nc,), 1e4, jnp.float32)])
    qb = sqxp.reshape(nblocks, _RQ)
    blo = qb[:, 0] - margin
    bhi = qb[:, -1] + margin
    clo = (jnp.searchsorted(scxp, blo, side='left') // _CH).astype(jnp.int32)
    chi = (-(-jnp.searchsorted(scxp, bhi, side='right') // _CH)
           ).astype(jnp.int32)
    chi = jnp.maximum(jnp.minimum(chi, nchunks), clo + 1)
    grid_spec = pltpu.PrefetchScalarGridSpec(
        num_scalar_prefetch=2,
        grid=(nblocks,),
        in_specs=[
            pl.BlockSpec((_RQ, 1), lambda i, *_: (i, 0)),
            pl.BlockSpec((_RQ, 1), lambda i, *_: (i, 0)),
            pl.BlockSpec((nchunks, 1, _CH), lambda i, *_: (0, 0, 0)),
            pl.BlockSpec((nchunks, 1, _CH), lambda i, *_: (0, 0, 0)),
        ],
        out_specs=pl.BlockSpec((_RQ, _KNN), lambda i, *_: (i, 0)),
    )
    idx = pl.pallas_call(
        functools.partial(_knn_body, nipad=ncp),
        grid_spec=grid_spec,
        out_shape=jax.ShapeDtypeStruct((nqp, _KNN), jnp.int32),
    )(clo, chi, sqxp.reshape(nqp, 1), sqyp.reshape(nqp, 1),
      scxp.reshape(nchunks, 1, _CH), scyp.reshape(nchunks, 1, _CH))
    return idx[:nq]


def _make_graph(q_pts, c_pts, margin):
    idx = _knn_pallas(q_pts[:, 0], q_pts[:, 1], c_pts[:, 0], c_pts[:, 1],
                      margin)
    idxt = idx.T
    nc = c_pts.shape[0]
    ncp = -(-nc // _CH) * _CH
    pad = jnp.full((ncp - nc,), 1e4, jnp.float32)
    cxp = jnp.concatenate([c_pts[:, 0], pad])
    cyp = jnp.concatenate([c_pts[:, 1], pad])
    cthp = jnp.concatenate([c_pts[:, 2], jnp.zeros_like(pad)])
    return (idxt, cxp[idxt], cyp[idxt], cthp[idxt],
            q_pts[:, 0][None], q_pts[:, 1][None], q_pts[:, 2][None])


def _do_conv(p, values_t, graph, crop, relu):
    idxt, xn, yn, thn, xo, yo, tho = graph
    nin = values_t.shape[1]
    vg = values_t[:, jnp.minimum(idxt, nin - 1)]
    return _conv_pallas(xo, yo, tho, xn, yn, thn, vg, p, crop, relu)


def kernel(values, input_points, points0, points1, params):
    radii = [0.2, 0.2 * (2.0 ** 0.5)]
    permi = jnp.argsort(input_points[:, 0])
    perm0 = jnp.argsort(points0[:, 0])
    perm1 = jnp.argsort(points1[:, 0])
    pis = input_points[permi]
    p0s = points0[perm0]
    p1s = points1[perm1]
    vt = values.T[:, permi]
    g1 = _make_graph(p0s, pis, radii[0] + 1e-3)
    v = _do_conv(params['first'], vt, g1, radii[0], True)
    g2 = _make_graph(p0s, p0s, radii[0] + 1e-3)
    g4 = None
    points_graphs = [g2, None]
    for i in range(2):
        g = points_graphs[i]
        if g is None:
            g = g4 = _make_graph(p1s, p1s, radii[1] + 1e-3)
        v_in = _bn_pallas(v, params['bn'][i][0])
        v = _do_conv(params['blocks'][i][0], v_in, g, radii[i], True)
        for j in range(1, 3):
            v = _bn_pallas(v, params['bn'][i][j])
            v = _do_conv(params['blocks'][i][j], v, g, radii[i], True)
        v = v + v_in
        if i < 1:
            v = _bn_pallas(v, params['bn'][i][3])
            g3 = _make_graph(p1s, p0s, radii[1] + 1e-3)
            v = _do_conv(params['down'][i], v, g3, radii[1], True)
    v = _bn_pallas(v, params['last_bn'])
    v = _do_conv(params['last'], v, g4, radii[-1], False)
    return jnp.mean(v, axis=1)


# speedup vs baseline: 1.6544x; 1.0037x over previous
import functools

import jax
import jax.numpy as jnp
from jax.experimental import pallas as pl

_KNN = 16
_NBASIS = 16


_R = 1024


def _pad_lanes(x, n_pad):
    pad = n_pad - x.shape[-1]
    if pad == 0:
        return x
    cfg = [(0, 0)] * (x.ndim - 1) + [(0, pad)]
    return jnp.pad(x, cfg)


def _bf(x):
    return x.astype(jnp.bfloat16).astype(jnp.float32)


def _conv_body(xo, yo, tho, xn, yn, thn, vg, w1t, b1, w2t, b2, wft, bo, out,
               *, crop, relu):
    c_in = vg.shape[0]
    R = out.shape[-1]
    xov = xo[...]
    yov = yo[...]
    thov = tho[...]
    co = jnp.cos(thov)
    so = jnp.sin(thov)
    dx = xn[...] - xov
    dy = yn[...] - yov
    rx = co * dx + so * dy
    ry = -so * dx + co * dy
    dth = thn[...] - thov
    rx = _bf(rx)
    ry = _bf(ry)
    cd = _bf(jnp.cos(dth))
    sd = _bf(jnp.sin(dth))
    b1v = b1[...]
    relcat = jnp.concatenate(
        [rx.reshape(1, _KNN * R), ry.reshape(1, _KNN * R),
         cd.reshape(1, _KNN * R), sd.reshape(1, _KNN * R)], axis=0)
    s1 = jnp.dot(w1t[...].astype(jnp.bfloat16), relcat.astype(jnp.bfloat16),
                 preferred_element_type=jnp.float32)
    h = jax.nn.relu(_bf(s1) + b1v)
    hf = h.astype(jnp.bfloat16)
    bas = jnp.dot(w2t[...].astype(jnp.bfloat16), hf,
                  preferred_element_type=jnp.float32)
    bas = bas + b2[...]
    bas = bas.reshape(_NBASIS, _KNN, R)
    dist2 = rx * rx + ry * ry
    mask = (jnp.sqrt(dist2) <= crop).astype(jnp.float32)
    bas = bas * mask[None]
    vgv = vg[...]
    if c_in > 1:
        bas = _bf(bas)
        vgv = _bf(vgv)
    g = jnp.zeros((_NBASIS, c_in, R), jnp.float32)
    for k in range(_KNN):
        g = g + bas[:, k, :][:, None, :] * vgv[:, k, :][None, :, :]
    gf = _bf(g).reshape(_NBASIS * c_in, R).astype(jnp.bfloat16)
    o = jnp.dot(wft[...].astype(jnp.bfloat16), gf,
                preferred_element_type=jnp.float32)
    o = o * (1.0 / _KNN) + bo[...]
    if relu:
        o = jax.nn.relu(o)
    out[...] = o


def _conv_pallas(xo, yo, tho, xn, yn, thn, vg, p, crop, relu):
    n = xo.shape[-1]
    c_in = vg.shape[0]
    c_out = p['W'].shape[-1]
    R = _R
    n_pad = -(-n // R) * R
    grid = n_pad // R
    xo, yo, tho, xn, yn, thn, vg = (
        _pad_lanes(a, n_pad) for a in (xo, yo, tho, xn, yn, thn, vg))
    wft = p['W'].reshape(_NBASIS * c_in, c_out).T
    w1t = p['W1'].T
    b1 = p['b1'].reshape(32, 1)
    b2 = p['b2'].reshape(_NBASIS, 1)
    bo = p['b'].reshape(c_out, 1)
    col = lambda i: (0, i)
    col3 = lambda i: (0, 0, i)
    full = lambda i: (0, 0)
    return pl.pallas_call(
        functools.partial(_conv_body, crop=crop, relu=relu),
        grid=(grid,),
        in_specs=[
            pl.BlockSpec((1, R), col),
            pl.BlockSpec((1, R), col),
            pl.BlockSpec((1, R), col),
            pl.BlockSpec((_KNN, R), col),
            pl.BlockSpec((_KNN, R), col),
            pl.BlockSpec((_KNN, R), col),
            pl.BlockSpec((c_in, _KNN, R), col3),
            pl.BlockSpec((32, 4), full),
            pl.BlockSpec((32, 1), full),
            pl.BlockSpec((_NBASIS, 32), full),
            pl.BlockSpec((_NBASIS, 1), full),
            pl.BlockSpec((c_out, _NBASIS * c_in), full),
            pl.BlockSpec((c_out, 1), full),
        ],
        out_specs=pl.BlockSpec((c_out, R), col),
        out_shape=jax.ShapeDtypeStruct((c_out, n_pad), jnp.float32),
    )(xo, yo, tho, xn, yn, thn, vg, w1t, b1, p['W2'].T, b2, wft, bo)[:, :n]


def _bn_body(x, gamma, beta, out):
    xv = x[...]
    m = jnp.mean(xv, axis=1, keepdims=True)
    v = jnp.mean((xv - m) ** 2, axis=1, keepdims=True)
    out[...] = (xv - m) / jnp.sqrt(v + 1e-5) * gamma[...] + beta[...]


def _bn_pallas(x, p):
    c, n = x.shape
    return pl.pallas_call(
        _bn_body,
        in_specs=[
            pl.BlockSpec((c, n), lambda: (0, 0)),
            pl.BlockSpec((c, 1), lambda: (0, 0)),
            pl.BlockSpec((c, 1), lambda: (0, 0)),
        ],
        out_specs=pl.BlockSpec((c, n), lambda: (0, 0)),
        out_shape=jax.ShapeDtypeStruct((c, n), jnp.float32),
    )(x, p['gamma'].reshape(c, 1), p['beta'].reshape(c, 1))


_RQ = 512
_CH = 512
_INF = 1e30
_BIGI = (1 << 30)


def _knn_body(clo_ref, chi_ref, qx, qy, cx, cy, out_i, *, nipad):
    b = pl.program_id(0)
    clo = clo_ref[b]
    chi = chi_ref[b]
    qxv = qx[...]
    qyv = qy[...]
    it = jax.lax.broadcasted_iota(jnp.int32, (_RQ, _CH), 1)
    it32 = jax.lax.broadcasted_iota(jnp.int32, (_RQ, 2 * _KNN), 1)

    def chunk_step(j, carry):
        best_d, best_i = carry
        cxv = cx[j]
        cyv = cy[j]
        dx = qxv - cxv
        dy = qyv - cyv
        d2 = dx * dx + dy * dy
        ms, gis = [], []
        for _ in range(_KNN):
            m = jnp.min(d2, axis=1, keepdims=True)
            pos = jnp.min(jnp.where(d2 == m, it, _BIGI), axis=1, keepdims=True)
            ms.append(m)
            gis.append(pos + j * _CH)
            d2 = jnp.where(it == pos, _INF, d2)
        cd = jnp.concatenate(ms, axis=1)
        ci = jnp.concatenate(gis, axis=1)
        md = jnp.concatenate([best_d, cd], axis=1)
        mi = jnp.concatenate([best_i, ci], axis=1)
        nd, ni = [], []
        for _ in range(_KNN):
            m = jnp.min(md, axis=1, keepdims=True)
            pos = jnp.min(jnp.where(md == m, it32, _BIGI), axis=1,
                          keepdims=True)
            sel = jnp.min(jnp.where(it32 == pos, mi, _BIGI), axis=1,
                          keepdims=True)
            nd.append(m)
            ni.append(sel)
            md = jnp.where(it32 == pos, _INF, md)
        return (jnp.concatenate(nd, axis=1), jnp.concatenate(ni, axis=1))

    init = (jnp.full((_RQ, _KNN), _INF, jnp.float32),
            jnp.full((_RQ, _KNN), nipad - 1, jnp.int32))
    best_d, best_i = jax.lax.fori_loop(clo, chi, chunk_step, init)
    out_i[...] = best_i


def _knn_pallas(sqx, sqy, scx, scy, margin):
    from jax.experimental.pallas import tpu as pltpu
    nq = sqx.shape[0]
    nc = scx.shape[0]
    nqp = -(-nq // _RQ) * _RQ
    ncp = -(-nc // _CH) * _CH
    nblocks = nqp // _RQ
    nchunks = ncp // _CH
    qpad = sqx[-1]
    sqxp = jnp.concatenate([sqx, jnp.full((nqp - nq,), qpad, jnp.float32)])
    sqyp = _pad_lanes(sqy, nqp)
    scxp = jnp.concatenate([scx, jnp.full((ncp - nc,), 1e4, jnp.float32)])
    scyp = jnp.concatenate([scy, jnp.full((ncp - nc,), 1e4, jnp.float32)])
    qb = sqxp.reshape(nblocks, _RQ)
    blo = qb[:, 0] - margin
    bhi = qb[:, -1] + margin
    clo = (jnp.searchsorted(scxp, blo, side='left') // _CH).astype(jnp.int32)
    chi = (-(-jnp.searchsorted(scxp, bhi, side='right') // _CH)
           ).astype(jnp.int32)
    chi = jnp.maximum(jnp.minimum(chi, nchunks), clo + 1)
    grid_spec = pltpu.PrefetchScalarGridSpec(
        num_scalar_prefetch=2,
        grid=(nblocks,),
        in_specs=[
            pl.BlockSpec((_RQ, 1), lambda i, *_: (i, 0)),
            pl.BlockSpec((_RQ, 1), lambda i, *_: (i, 0)),
            pl.BlockSpec((nchunks, 1, _CH), lambda i, *_: (0, 0, 0)),
            pl.BlockSpec((nchunks, 1, _CH), lambda i, *_: (0, 0, 0)),
        ],
        out_specs=pl.BlockSpec((_RQ, _KNN), lambda i, *_: (i, 0)),
    )
    idx = pl.pallas_call(
        functools.partial(_knn_body, nipad=ncp),
        grid_spec=grid_spec,
        out_shape=jax.ShapeDtypeStruct((nqp, _KNN), jnp.int32),
    )(clo, chi, sqxp.reshape(nqp, 1), sqyp.reshape(nqp, 1),
      scxp.reshape(nchunks, 1, _CH), scyp.reshape(nchunks, 1, _CH))
    return idx[:nq]


def _make_graph(q_pts, c_pts, margin):
    idx = _knn_pallas(q_pts[:, 0], q_pts[:, 1], c_pts[:, 0], c_pts[:, 1],
                      margin)
    idxt = idx.T
    nc = c_pts.shape[0]
    ncp = -(-nc // _CH) * _CH
    pad = jnp.full((ncp - nc,), 1e4, jnp.float32)
    cxp = jnp.concatenate([c_pts[:, 0], pad])
    cyp = jnp.concatenate([c_pts[:, 1], pad])
    cthp = jnp.concatenate([c_pts[:, 2], jnp.zeros_like(pad)])
    return (idxt, cxp[idxt], cyp[idxt], cthp[idxt],
            q_pts[:, 0][None], q_pts[:, 1][None], q_pts[:, 2][None])


def _do_conv(p, values_t, graph, crop, relu):
    idxt, xn, yn, thn, xo, yo, tho = graph
    nin = values_t.shape[1]
    vg = values_t[:, jnp.minimum(idxt, nin - 1)]
    return _conv_pallas(xo, yo, tho, xn, yn, thn, vg, p, crop, relu)


def kernel(values, input_points, points0, points1, params):
    radii = [0.2, 0.2 * (2.0 ** 0.5)]
    permi = jnp.argsort(input_points[:, 0])
    perm0 = jnp.argsort(points0[:, 0])
    perm1 = jnp.argsort(points1[:, 0])
    pis = input_points[permi]
    p0s = points0[perm0]
    p1s = points1[perm1]
    vt = values.T[:, permi]
    g1 = _make_graph(p0s, pis, radii[0] + 1e-3)
    v = _do_conv(params['first'], vt, g1, radii[0], True)
    g2 = _make_graph(p0s, p0s, radii[0] + 1e-3)
    g4 = None
    points_graphs = [g2, None]
    for i in range(2):
        g = points_graphs[i]
        if g is None:
            g = g4 = _make_graph(p1s, p1s, radii[1] + 1e-3)
        v_in = _bn_pallas(v, params['bn'][i][0])
        v = _do_conv(params['blocks'][i][0], v_in, g, radii[i], True)
        for j in range(1, 3):
            v = _bn_pallas(v, params['bn'][i][j])
            v = _do_conv(params['blocks'][i][j], v, g, radii[i], True)
        v = v + v_in
        if i < 1:
            v = _bn_pallas(v, params['bn'][i][3])
            g3 = _make_graph(p1s, p0s, radii[1] + 1e-3)
            v = _do_conv(params['down'][i], v, g3, radii[1], True)
    v = _bn_pallas(v, params['last_bn'])
    v = _do_conv(params['last'], v, g4, radii[-1], False)
    return jnp.mean(v, axis=1)
